# Initial kernel scaffold; baseline (speedup 1.0000x reference)
import jax
import jax.numpy as jnp
from jax.experimental import pallas as pl


def kernel(inputs, alpha, targets, e):
    raise NotImplementedError("write your pallas kernel here")


# speedup vs baseline: 4.3405x; 4.3405x over previous
import jax
import jax.numpy as jnp
from jax import lax
from jax.experimental import pallas as pl

_N = 16384
_C = 100
_BN = 1024


def _loss_kernel(x_ref, t_ref, a_ref, acc_ref):
    x = x_ref[...]
    t = t_ref[...]
    alpha_row = a_ref[...]

    col = lax.broadcasted_iota(jnp.int32, x.shape, 1)
    onehot = col == t

    m = jnp.max(x, axis=1, keepdims=True)
    s = jnp.sum(jnp.exp(x - m), axis=1, keepdims=True)
    g = jnp.sum(jnp.where(onehot, x, 0.0), axis=1, keepdims=True)
    a = jnp.sum(jnp.where(onehot, alpha_row, 0.0), axis=1, keepdims=True)

    p = jnp.exp(g - m) / s + 1e-05
    lg = jnp.log(p)
    d = 0.5 - p
    q = 1.0 - p
    per_row = a * (d * d * d * lg * lg + 0.01 + q * q)
    partial = (jnp.sum(per_row) * (1.0 / _N)).reshape(1, 1)

    @pl.when(pl.program_id(0) == 0)
    def _init():
        acc_ref[...] = jnp.zeros_like(acc_ref)

    acc_ref[...] += partial


def kernel(inputs, alpha, targets, e):
    del e
    t2 = targets.reshape(_N, 1)
    alpha_row = alpha.reshape(1, _C)

    acc = pl.pallas_call(
        _loss_kernel,
        grid=(_N // _BN,),
        in_specs=[
            pl.BlockSpec((_BN, _C), lambda i: (i, 0)),
            pl.BlockSpec((_BN, 1), lambda i: (i, 0)),
            pl.BlockSpec((1, _C), lambda i: (0, 0)),
        ],
        out_specs=pl.BlockSpec((1, 1), lambda i: (0, 0)),
        out_shape=jax.ShapeDtypeStruct((1, 1), jnp.float32),
    )(inputs, t2, alpha_row)
    return acc[0, 0]
